# baseline (speedup 1.0000x reference)
import functools

import jax
import jax.numpy as jnp
from jax import lax
from jax.experimental import pallas as pl
from jax.experimental.pallas import tpu as pltpu
from jax.experimental.pallas import tpu_sc as plsc

N = 10000
D = 128
E = 320000
NC = 2
NS = 16
K = 128
EPC = E // NC
EPT_M = 9984
MQ = 76
LEFT0 = NS * EPT_M
RPT = 624
TAIL = N - RPT * NS
NREL = 7
CW = 16


def _sc_body(xv, xe, xf, e0r, e1r, e2r, e3r, e4r, e5r, e6r, zf, zch,
             s_out, c_out,
             acc, cacc, rows0, rows1, i20, i21, i22, i23, ones,
             sem_g0, sem_g1, sem_i0, sem_i1, sem_s0, sem_s1, sem_z):
    c = lax.axis_index("c")
    s = lax.axis_index("s")
    one0 = jnp.where(lax.iota(jnp.int32, 16) == 0, 1.0, 0.0)
    rows = (rows0, rows1)
    i2 = (i20, i21, i22, i23)
    sem_g = (sem_g0, sem_g1)
    sem_i = (sem_i0, sem_i1)
    sem_s = (sem_s0, sem_s1)

    def init_ones(i, carry):
        ones[i, :] = one0
        return carry

    lax.fori_loop(0, K, init_ones, 0)

    tables = [xv, xv, xv, xe, xe, xf, xf]
    edges = [e0r, e1r, e2r, e3r, e4r, e5r, e6r]
    row0 = s * RPT
    ebase = c * EPC + s * EPT_M

    def zero_accs():
        zcopies = [(zf.at[pl.ds(0, RPT), :], acc.at[pl.ds(row0, RPT), :]),
                   (zch.at[pl.ds(0, RPT), :], cacc.at[pl.ds(row0, RPT), :])]
        tcopies = [(zf.at[pl.ds(0, TAIL), :],
                    acc.at[pl.ds(RPT * NS, TAIL), :]),
                   (zch.at[pl.ds(0, TAIL), :],
                    cacc.at[pl.ds(RPT * NS, TAIL), :])]
        for src, dst in zcopies:
            pltpu.async_copy(src, dst, sem_z)

        @pl.when(s == 0)
        def _():
            for src, dst in tcopies:
                pltpu.async_copy(src, dst, sem_z)

        for src, dst in zcopies:
            pltpu.make_async_copy(src, dst, sem_z).wait()

        @pl.when(s == 0)
        def _():
            for src, dst in tcopies:
                pltpu.make_async_copy(src, dst, sem_z).wait()

    def wait_scat(rb):
        pltpu.make_async_copy(rows[rb], acc.at[i20.at[1]], sem_s[rb]).wait()
        pltpu.make_async_copy(ones, cacc.at[i20.at[1]], sem_s[rb]).wait()

    def scat(rb, db):
        pltpu.async_copy(rows[rb], acc.at[i2[db].at[1]], sem_s[rb],
                         add=True)
        pltpu.async_copy(ones, cacc.at[i2[db].at[1]], sem_s[rb], add=True)

    def two_chunks(table, ei, base0, base1):
        pltpu.sync_copy(ei.at[:, pl.ds(base0, K)], i20)
        pltpu.async_copy(table.at[i20.at[0]], rows0, sem_g0)
        pltpu.sync_copy(ei.at[:, pl.ds(base1, K)], i21)
        pltpu.make_async_copy(table.at[i20.at[0]], rows0, sem_g0).wait()
        pltpu.async_copy(table.at[i21.at[0]], rows1, sem_g1)
        scat(0, 0)
        pltpu.make_async_copy(table.at[i21.at[0]], rows1, sem_g1).wait()
        scat(1, 1)
        wait_scat(0)
        wait_scat(1)

    def prologue(r):
        table = tables[r]
        ei = edges[r]
        pltpu.sync_copy(ei.at[:, pl.ds(ebase, K)], i20)
        pltpu.async_copy(table.at[i20.at[0]], rows[0], sem_g[0])
        pltpu.async_copy(ei.at[:, pl.ds(ebase + K, K)], i21, sem_i[1])

    zero_accs()
    prologue(0)

    for r in range(NREL):
        table = tables[r]
        ei = edges[r]
        e0 = ebase

        def wait_idx(rb, ei=ei):
            pltpu.make_async_copy(
                ei.at[:, pl.ds(e0, K)], i20, sem_i[rb]).wait()

        plsc.subcore_barrier()


        def quad(p, carry):
            for b4 in range(4):
                g = p * 4 + b4
                rb = b4 % 2
                ob = 1 - rb
                pltpu.make_async_copy(
                    table.at[i2[b4].at[0]], rows[rb], sem_g[rb]).wait()

                @pl.when((g + 1 < MQ) & (g >= 1))
                def _():
                    wait_scat(ob)

                @pl.when(g + 1 < MQ)
                def _():
                    wait_idx(ob)
                    pltpu.async_copy(table.at[i2[(b4 + 1) % 4].at[0]],
                                     rows[ob], sem_g[ob])

                scat(rb, b4)

                @pl.when(g + 2 < MQ)
                def _():
                    b2 = e0 + (g + 2) * K
                    pltpu.async_copy(ei.at[:, pl.ds(b2, K)],
                                     i2[(b4 + 2) % 4], sem_i[rb])
            return carry

        lax.fori_loop(0, MQ // 4, quad, 0)
        wait_scat(0)
        wait_scat(1)

        two_chunks(table, ei, e0 + MQ * K, e0 + (MQ + 1) * K)

        @pl.when(s == 0)
        def _leftover():
            bl = c * EPC + LEFT0
            two_chunks(table, ei, bl, bl + K)

        if r + 1 < NREL:
            prologue(r + 1)

        plsc.subcore_barrier()

        fcopies = [(acc.at[pl.ds(row0, RPT), :],
                    s_out.at[r, c, pl.ds(row0, RPT), :]),
                   (cacc.at[pl.ds(row0, RPT), :],
                    c_out.at[r, c, pl.ds(row0, RPT), :])]
        ftail = [(acc.at[pl.ds(RPT * NS, TAIL), :],
                  s_out.at[r, c, pl.ds(RPT * NS, TAIL), :]),
                 (cacc.at[pl.ds(RPT * NS, TAIL), :],
                  c_out.at[r, c, pl.ds(RPT * NS, TAIL), :])]
        for src, dst in fcopies:
            pltpu.async_copy(src, dst, sem_z)

        @pl.when(s == 0)
        def _flush_tail():
            for src, dst in ftail:
                pltpu.async_copy(src, dst, sem_z)

        for src, dst in fcopies:
            pltpu.make_async_copy(src, dst, sem_z).wait()

        @pl.when(s == 0)
        def _flush_tail_wait():
            for src, dst in ftail:
                pltpu.make_async_copy(src, dst, sem_z).wait()

        if r + 1 < NREL:
            zero_accs()


_sc_segment_sums = functools.partial(
    pl.kernel,
    out_type=(
        jax.ShapeDtypeStruct((NREL, NC, N, D), jnp.float32),
        jax.ShapeDtypeStruct((NREL, NC, N, CW), jnp.float32),
    ),
    mesh=plsc.VectorSubcoreMesh(
        core_axis_name="c", subcore_axis_name="s",
        num_cores=NC, num_subcores=NS),
    scratch_types=(
        [pltpu.VMEM_SHARED((N, D), jnp.float32),
         pltpu.VMEM_SHARED((N, CW), jnp.float32)]
        + [pltpu.VMEM((K, D), jnp.float32)] * 2
        + [pltpu.VMEM((2, K), jnp.int32)] * 4
        + [pltpu.VMEM((K, CW), jnp.float32)]
        + [pltpu.SemaphoreType.DMA] * 7
    ),
    compiler_params=pltpu.CompilerParams(use_tc_tiling_on_sc=False),
)(_sc_body)


def _tc_combine(x, a, b, s_part, c_part, rs, wls):
    nrel = len(rs)
    blk = 1000
    grid = (N // blk,)

    def body(*refs):
        x_ref = refs[0]
        a_ref = refs[1]
        b_ref = refs[2]
        out_ref = refs[3 + 5 * nrel]
        out = jnp.dot(x_ref[...], a_ref[...],
                      preferred_element_type=jnp.float32) + b_ref[...]
        for i in range(nrel):
            s0 = refs[3 + 4 * i][0, 0]
            s1 = refs[4 + 4 * i][0, 0]
            c0 = refs[5 + 4 * i][0, 0]
            c1 = refs[6 + 4 * i][0, 0]
            wl = refs[3 + 4 * nrel + i][...]
            cnt = c0[:, 0:1] + c1[:, 0:1]
            mean = (s0 + s1) * (1.0 / jnp.maximum(cnt, 1.0))
            out = out + jnp.dot(mean, wl, preferred_element_type=jnp.float32)
        out_ref[...] = jnp.maximum(out, 0.0)

    row_spec = pl.BlockSpec((blk, D), lambda i: (i, 0))
    w_spec = pl.BlockSpec((D, D), lambda i: (0, 0))
    b_spec = pl.BlockSpec((1, D), lambda i: (0, 0))
    in_specs = [row_spec, w_spec, b_spec]
    operands = [x, a, b]
    for r in rs:
        for cc in (0, 1):
            in_specs.append(pl.BlockSpec(
                (1, 1, blk, D), lambda i, r=r, cc=cc: (r, cc, i, 0)))
            operands.append(s_part)
        for cc in (0, 1):
            in_specs.append(pl.BlockSpec(
                (1, 1, blk, CW), lambda i, r=r, cc=cc: (r, cc, i, 0)))
            operands.append(c_part)
    in_specs += [w_spec] * nrel
    operands += list(wls)
    return pl.pallas_call(
        body,
        grid=grid,
        in_specs=in_specs,
        out_specs=row_spec,
        out_shape=jax.ShapeDtypeStruct((N, D), jnp.float32),
    )(*operands)


def kernel(x_v, x_e, x_f,
           ei_vv, wl_vv, bl_vv, wr_vv,
           ei_ve, wl_ve, bl_ve, wr_ve,
           ei_vf, wl_vf, bl_vf, wr_vf,
           ei_ev, wl_ev, bl_ev, wr_ev,
           ei_ef, wl_ef, bl_ef, wr_ef,
           ei_fv, wl_fv, bl_fv, wr_fv,
           ei_fe, wl_fe, bl_fe, wr_fe,
           w_self_v, b_self_v,
           w_self_e, b_self_e,
           w_self_f, b_self_f):
    zf = jnp.zeros((RPT + 16, D), jnp.float32)
    zch = jnp.zeros((RPT + 16, CW), jnp.float32)
    s_part, c_part = _sc_segment_sums(
        x_v, x_e, x_f, ei_vv, ei_ve, ei_vf, ei_ev, ei_ef, ei_fv, ei_fe,
        zf, zch)

    a_v = w_self_v + (wr_vv + wr_ev + wr_fv) / 3.0
    b_v = (b_self_v + (bl_vv + bl_ev + bl_fv) / 3.0).reshape(1, D)
    a_e = w_self_e + (wr_ve + wr_fe) / 2.0
    b_e = (b_self_e + (bl_ve + bl_fe) / 2.0).reshape(1, D)
    a_f = w_self_f + (wr_vf + wr_ef) / 2.0
    b_f = (b_self_f + (bl_vf + bl_ef) / 2.0).reshape(1, D)

    xv = _tc_combine(x_v, a_v, b_v, s_part, c_part, [0, 3, 5],
                     [wl_vv / 3.0, wl_ev / 3.0, wl_fv / 3.0])
    xe = _tc_combine(x_e, a_e, b_e, s_part, c_part, [1, 6],
                     [wl_ve / 2.0, wl_fe / 2.0])
    xf = _tc_combine(x_f, a_f, b_f, s_part, c_part, [2, 4],
                     [wl_vf / 2.0, wl_ef / 2.0])
    return xv, xe, xf


# speedup vs baseline: 1.0037x; 1.0037x over previous
import functools

import jax
import jax.numpy as jnp
from jax import lax
from jax.experimental import pallas as pl
from jax.experimental.pallas import tpu as pltpu
from jax.experimental.pallas import tpu_sc as plsc

N = 10000
D = 128
E = 320000
NC = 2
NS = 16
K = 128
EPC = E // NC
EPT_M = 9984
MQ = 76
LEFT0 = NS * EPT_M
RPT = 624
TAIL = N - RPT * NS
NREL = 7
CW = 16


def _sc_body(xv, xe, xf, e0r, e1r, e2r, e3r, e4r, e5r, e6r, zf, zch,
             s_out, c_out,
             acc, cacc, rows0, rows1, i20, i21, i22, i23, ones,
             sem_g0, sem_g1, sem_i0, sem_i1, sem_s0, sem_s1, sem_z):
    c = lax.axis_index("c")
    s = lax.axis_index("s")
    one0 = jnp.where(lax.iota(jnp.int32, 16) == 0, 1.0, 0.0)
    rows = (rows0, rows1)
    i2 = (i20, i21, i22, i23)
    sem_g = (sem_g0, sem_g1)
    sem_i = (sem_i0, sem_i1)
    sem_s = (sem_s0, sem_s1)

    def init_ones(i, carry):
        ones[i, :] = one0
        return carry

    lax.fori_loop(0, K, init_ones, 0)

    tables = [xv, xv, xv, xe, xe, xf, xf]
    edges = [e0r, e1r, e2r, e3r, e4r, e5r, e6r]
    row0 = s * RPT
    ebase = c * EPC + s * EPT_M

    def zero_accs():
        zcopies = [(zf.at[pl.ds(0, RPT), :], acc.at[pl.ds(row0, RPT), :]),
                   (zch.at[pl.ds(0, RPT), :], cacc.at[pl.ds(row0, RPT), :])]
        tcopies = [(zf.at[pl.ds(0, TAIL), :],
                    acc.at[pl.ds(RPT * NS, TAIL), :]),
                   (zch.at[pl.ds(0, TAIL), :],
                    cacc.at[pl.ds(RPT * NS, TAIL), :])]
        for src, dst in zcopies:
            pltpu.async_copy(src, dst, sem_z)

        @pl.when(s == 0)
        def _():
            for src, dst in tcopies:
                pltpu.async_copy(src, dst, sem_z)

        for src, dst in zcopies:
            pltpu.make_async_copy(src, dst, sem_z).wait()

        @pl.when(s == 0)
        def _():
            for src, dst in tcopies:
                pltpu.make_async_copy(src, dst, sem_z).wait()

    def wait_scat(rb):
        pltpu.make_async_copy(rows[rb], acc.at[i20.at[1]], sem_s[rb]).wait()
        pltpu.make_async_copy(ones, cacc.at[i20.at[1]], sem_s[rb]).wait()

    def scat(rb, db):
        pltpu.async_copy(rows[rb], acc.at[i2[db].at[1]], sem_s[rb],
                         add=True)
        pltpu.async_copy(ones, cacc.at[i2[db].at[1]], sem_s[rb], add=True)

    def two_chunks(table, ei, base0, base1):
        pltpu.sync_copy(ei.at[:, pl.ds(base0, K)], i20)
        pltpu.async_copy(table.at[i20.at[0]], rows0, sem_g0)
        pltpu.sync_copy(ei.at[:, pl.ds(base1, K)], i21)
        pltpu.make_async_copy(table.at[i20.at[0]], rows0, sem_g0).wait()
        pltpu.async_copy(table.at[i21.at[0]], rows1, sem_g1)
        scat(0, 0)
        pltpu.make_async_copy(table.at[i21.at[0]], rows1, sem_g1).wait()
        scat(1, 1)
        wait_scat(0)
        wait_scat(1)

    def prologue(r):
        table = tables[r]
        ei = edges[r]
        pltpu.sync_copy(ei.at[:, pl.ds(ebase, K)], i20)
        pltpu.async_copy(table.at[i20.at[0]], rows[0], sem_g[0])
        pltpu.async_copy(ei.at[:, pl.ds(ebase + K, K)], i21, sem_i[1])

    zero_accs()
    prologue(0)

    for r in range(NREL):
        table = tables[r]
        ei = edges[r]
        e0 = ebase

        def wait_idx(rb, ei=ei):
            pltpu.make_async_copy(
                ei.at[:, pl.ds(e0, K)], i20, sem_i[rb]).wait()

        plsc.subcore_barrier()


        def quad(p, carry):
            for b4 in range(4):
                g = p * 4 + b4
                rb = b4 % 2
                ob = 1 - rb
                pltpu.make_async_copy(
                    table.at[i2[b4].at[0]], rows[rb], sem_g[rb]).wait()

                @pl.when((g + 1 < MQ) & (g >= 1))
                def _():
                    wait_scat(ob)

                @pl.when(g + 1 < MQ)
                def _():
                    wait_idx(ob)
                    pltpu.async_copy(table.at[i2[(b4 + 1) % 4].at[0]],
                                     rows[ob], sem_g[ob])

                scat(rb, b4)

                @pl.when(g + 2 < MQ)
                def _():
                    b2 = e0 + (g + 2) * K
                    pltpu.async_copy(ei.at[:, pl.ds(b2, K)],
                                     i2[(b4 + 2) % 4], sem_i[rb])
            return carry

        lax.fori_loop(0, MQ // 4, quad, 0)
        wait_scat(0)
        wait_scat(1)

        two_chunks(table, ei, e0 + MQ * K, e0 + (MQ + 1) * K)

        @pl.when(s == 0)
        def _leftover():
            bl = c * EPC + LEFT0
            two_chunks(table, ei, bl, bl + K)

        if r + 1 < NREL:
            prologue(r + 1)

        plsc.subcore_barrier()

        fcopies = [(acc.at[pl.ds(row0, RPT), :],
                    s_out.at[r, c, pl.ds(row0, RPT), :]),
                   (cacc.at[pl.ds(row0, RPT), :],
                    c_out.at[r, c, pl.ds(row0, RPT), :])]
        ftail = [(acc.at[pl.ds(RPT * NS, TAIL), :],
                  s_out.at[r, c, pl.ds(RPT * NS, TAIL), :]),
                 (cacc.at[pl.ds(RPT * NS, TAIL), :],
                  c_out.at[r, c, pl.ds(RPT * NS, TAIL), :])]
        for src, dst in fcopies:
            pltpu.async_copy(src, dst, sem_z)

        @pl.when(s == 0)
        def _flush_tail():
            for src, dst in ftail:
                pltpu.async_copy(src, dst, sem_z)

        for src, dst in fcopies:
            pltpu.make_async_copy(src, dst, sem_z).wait()

        @pl.when(s == 0)
        def _flush_tail_wait():
            for src, dst in ftail:
                pltpu.make_async_copy(src, dst, sem_z).wait()

        if r + 1 < NREL:
            zero_accs()


_sc_segment_sums = functools.partial(
    pl.kernel,
    out_type=(
        jax.ShapeDtypeStruct((NREL, NC, N, D), jnp.float32),
        jax.ShapeDtypeStruct((NREL, NC, N, CW), jnp.float32),
    ),
    mesh=plsc.VectorSubcoreMesh(
        core_axis_name="c", subcore_axis_name="s",
        num_cores=NC, num_subcores=NS),
    scratch_types=(
        [pltpu.VMEM_SHARED((N, D), jnp.float32),
         pltpu.VMEM_SHARED((N, CW), jnp.float32)]
        + [pltpu.VMEM((K, D), jnp.float32)] * 2
        + [pltpu.VMEM((2, K), jnp.int32)] * 4
        + [pltpu.VMEM((K, CW), jnp.float32)]
        + [pltpu.SemaphoreType.DMA] * 7
    ),
    compiler_params=pltpu.CompilerParams(use_tc_tiling_on_sc=False),
)(_sc_body)


def _tc_combine(x, a, b, s_part, c_part, rs, wls):
    nrel = len(rs)
    blk = 1000
    ngrid = N // blk

    def body(*refs):
        x_ref = refs[0]
        a_ref = refs[1]
        b_ref = refs[2]
        c_ref = refs[3 + 3 * nrel]
        out_ref = refs[4 + 3 * nrel]
        cbuf = refs[5 + 3 * nrel]
        csem = refs[6 + 3 * nrel]
        i = pl.program_id(0)

        def fetch(gi, pb):
            for ri, r in enumerate(rs):
                for cc in (0, 1):
                    pltpu.make_async_copy(
                        c_ref.at[r, cc, pl.ds(gi * blk, blk), :],
                        cbuf.at[pb, ri, cc], csem).start()

        def drain(gi, pb):
            for ri, r in enumerate(rs):
                for cc in (0, 1):
                    pltpu.make_async_copy(
                        c_ref.at[r, cc, pl.ds(gi * blk, blk), :],
                        cbuf.at[pb, ri, cc], csem).wait()

        @pl.when(i == 0)
        def _():
            fetch(0, 0)

        drain(i, lax.rem(i, 2))

        @pl.when(i + 1 < ngrid)
        def _():
            fetch(i + 1, lax.rem(i + 1, 2))

        out = jnp.dot(x_ref[...], a_ref[...],
                      preferred_element_type=jnp.float32) + b_ref[...]
        pb = lax.rem(i, 2)
        for ri in range(nrel):
            s0 = refs[3 + 2 * ri][0, 0]
            s1 = refs[4 + 2 * ri][0, 0]
            wl = refs[3 + 2 * nrel + ri][...]
            cnt = cbuf[pb, ri, 0, :, 0:1] + cbuf[pb, ri, 1, :, 0:1]
            mean = (s0 + s1) * (1.0 / jnp.maximum(cnt, 1.0))
            out = out + jnp.dot(mean, wl, preferred_element_type=jnp.float32)
        out_ref[...] = jnp.maximum(out, 0.0)

    row_spec = pl.BlockSpec((blk, D), lambda i: (i, 0))
    w_spec = pl.BlockSpec((D, D), lambda i: (0, 0))
    b_spec = pl.BlockSpec((1, D), lambda i: (0, 0))
    in_specs = [row_spec, w_spec, b_spec]
    operands = [x, a, b]
    for r in rs:
        for cc in (0, 1):
            in_specs.append(pl.BlockSpec(
                (1, 1, blk, D), lambda i, r=r, cc=cc: (r, cc, i, 0)))
            operands.append(s_part)
    in_specs += [w_spec] * nrel
    operands += list(wls)
    in_specs.append(pl.BlockSpec(memory_space=pl.ANY))
    operands.append(c_part)
    return pl.pallas_call(
        body,
        grid=(ngrid,),
        in_specs=in_specs,
        out_specs=row_spec,
        out_shape=jax.ShapeDtypeStruct((N, D), jnp.float32),
        scratch_shapes=[
            pltpu.VMEM((2, nrel, 2, blk, CW), jnp.float32),
            pltpu.SemaphoreType.DMA,
        ],
    )(*operands)


def kernel(x_v, x_e, x_f,
           ei_vv, wl_vv, bl_vv, wr_vv,
           ei_ve, wl_ve, bl_ve, wr_ve,
           ei_vf, wl_vf, bl_vf, wr_vf,
           ei_ev, wl_ev, bl_ev, wr_ev,
           ei_ef, wl_ef, bl_ef, wr_ef,
           ei_fv, wl_fv, bl_fv, wr_fv,
           ei_fe, wl_fe, bl_fe, wr_fe,
           w_self_v, b_self_v,
           w_self_e, b_self_e,
           w_self_f, b_self_f):
    zf = jnp.zeros((RPT + 16, D), jnp.float32)
    zch = jnp.zeros((RPT + 16, CW), jnp.float32)
    s_part, c_part = _sc_segment_sums(
        x_v, x_e, x_f, ei_vv, ei_ve, ei_vf, ei_ev, ei_ef, ei_fv, ei_fe,
        zf, zch)

    a_v = w_self_v + (wr_vv + wr_ev + wr_fv) / 3.0
    b_v = (b_self_v + (bl_vv + bl_ev + bl_fv) / 3.0).reshape(1, D)
    a_e = w_self_e + (wr_ve + wr_fe) / 2.0
    b_e = (b_self_e + (bl_ve + bl_fe) / 2.0).reshape(1, D)
    a_f = w_self_f + (wr_vf + wr_ef) / 2.0
    b_f = (b_self_f + (bl_vf + bl_ef) / 2.0).reshape(1, D)

    xv = _tc_combine(x_v, a_v, b_v, s_part, c_part, [0, 3, 5],
                     [wl_vv / 3.0, wl_ev / 3.0, wl_fv / 3.0])
    xe = _tc_combine(x_e, a_e, b_e, s_part, c_part, [1, 6],
                     [wl_ve / 2.0, wl_fe / 2.0])
    xf = _tc_combine(x_f, a_f, b_f, s_part, c_part, [2, 4],
                     [wl_vf / 2.0, wl_ef / 2.0])
    return xv, xe, xf


# speedup vs baseline: 1.0170x; 1.0132x over previous
import functools

import jax
import jax.numpy as jnp
from jax import lax
from jax.experimental import pallas as pl
from jax.experimental.pallas import tpu as pltpu
from jax.experimental.pallas import tpu_sc as plsc

N = 10000
D = 128
E = 320000
NC = 2
NS = 16
K = 128
EPC = E // NC
EPT_M = 9984
MQ = 76
LEFT0 = NS * EPT_M
RPT = 624
TAIL = N - RPT * NS
NREL = 7
CW = 16


def _sc_body(srcs, *args):
    nr = len(srcs)
    xv, xe, xf = args[0:3]
    es = args[3:3 + nr]
    zf, zch, s_out, c_out = args[3 + nr:7 + nr]
    (acc, cacc, rows0, rows1, i20, i21, i22, i23, ones,
     sem_g0, sem_g1, sem_i0, sem_i1, sem_s0, sem_s1, sem_z) = args[7 + nr:]
    c = lax.axis_index("c")
    s = lax.axis_index("s")
    one0 = jnp.where(lax.iota(jnp.int32, 16) == 0, 1.0, 0.0)
    rows = (rows0, rows1)
    i2 = (i20, i21, i22, i23)
    sem_g = (sem_g0, sem_g1)
    sem_i = (sem_i0, sem_i1)
    sem_s = (sem_s0, sem_s1)

    def init_ones(i, carry):
        ones[i, :] = one0
        return carry

    lax.fori_loop(0, K, init_ones, 0)

    tables = [(xv, xe, xf)[t] for t in srcs]
    edges = list(es)
    row0 = s * RPT
    ebase = c * EPC + s * EPT_M

    def zero_accs():
        zcopies = [(zf.at[pl.ds(0, RPT), :], acc.at[pl.ds(row0, RPT), :]),
                   (zch.at[pl.ds(0, RPT), :], cacc.at[pl.ds(row0, RPT), :])]
        tcopies = [(zf.at[pl.ds(0, TAIL), :],
                    acc.at[pl.ds(RPT * NS, TAIL), :]),
                   (zch.at[pl.ds(0, TAIL), :],
                    cacc.at[pl.ds(RPT * NS, TAIL), :])]
        for src, dst in zcopies:
            pltpu.async_copy(src, dst, sem_z)

        @pl.when(s == 0)
        def _():
            for src, dst in tcopies:
                pltpu.async_copy(src, dst, sem_z)

        for src, dst in zcopies:
            pltpu.make_async_copy(src, dst, sem_z).wait()

        @pl.when(s == 0)
        def _():
            for src, dst in tcopies:
                pltpu.make_async_copy(src, dst, sem_z).wait()

    def wait_scat(rb):
        pltpu.make_async_copy(rows[rb], acc.at[i20.at[1]], sem_s[rb]).wait()
        pltpu.make_async_copy(ones, cacc.at[i20.at[1]], sem_s[rb]).wait()

    def scat(rb, db):
        pltpu.async_copy(rows[rb], acc.at[i2[db].at[1]], sem_s[rb],
                         add=True)
        pltpu.async_copy(ones, cacc.at[i2[db].at[1]], sem_s[rb], add=True)

    def two_chunks(table, ei, base0, base1):
        pltpu.sync_copy(ei.at[:, pl.ds(base0, K)], i20)
        pltpu.async_copy(table.at[i20.at[0]], rows0, sem_g0)
        pltpu.sync_copy(ei.at[:, pl.ds(base1, K)], i21)
        pltpu.make_async_copy(table.at[i20.at[0]], rows0, sem_g0).wait()
        pltpu.async_copy(table.at[i21.at[0]], rows1, sem_g1)
        scat(0, 0)
        pltpu.make_async_copy(table.at[i21.at[0]], rows1, sem_g1).wait()
        scat(1, 1)
        wait_scat(0)
        wait_scat(1)

    def prologue(r):
        table = tables[r]
        ei = edges[r]
        pltpu.sync_copy(ei.at[:, pl.ds(ebase, K)], i20)
        pltpu.async_copy(table.at[i20.at[0]], rows[0], sem_g[0])
        pltpu.async_copy(ei.at[:, pl.ds(ebase + K, K)], i21, sem_i[1])

    zero_accs()
    prologue(0)

    for r in range(nr):
        table = tables[r]
        ei = edges[r]
        e0 = ebase

        def wait_idx(rb, ei=ei):
            pltpu.make_async_copy(
                ei.at[:, pl.ds(e0, K)], i20, sem_i[rb]).wait()

        plsc.subcore_barrier()


        def quad(p, carry):
            for b4 in range(4):
                g = p * 4 + b4
                rb = b4 % 2
                ob = 1 - rb
                pltpu.make_async_copy(
                    table.at[i2[b4].at[0]], rows[rb], sem_g[rb]).wait()

                @pl.when((g + 1 < MQ) & (g >= 1))
                def _():
                    wait_scat(ob)

                @pl.when(g + 1 < MQ)
                def _():
                    wait_idx(ob)
                    pltpu.async_copy(table.at[i2[(b4 + 1) % 4].at[0]],
                                     rows[ob], sem_g[ob])

                scat(rb, b4)

                @pl.when(g + 2 < MQ)
                def _():
                    b2 = e0 + (g + 2) * K
                    pltpu.async_copy(ei.at[:, pl.ds(b2, K)],
                                     i2[(b4 + 2) % 4], sem_i[rb])
            return carry

        lax.fori_loop(0, MQ // 4, quad, 0)
        wait_scat(0)
        wait_scat(1)

        two_chunks(table, ei, e0 + MQ * K, e0 + (MQ + 1) * K)

        @pl.when(s == 0)
        def _leftover():
            bl = c * EPC + LEFT0
            two_chunks(table, ei, bl, bl + K)

        if r + 1 < nr:
            prologue(r + 1)

        plsc.subcore_barrier()

        fcopies = [(acc.at[pl.ds(row0, RPT), :],
                    s_out.at[r, c, pl.ds(row0, RPT), :]),
                   (cacc.at[pl.ds(row0, RPT), :],
                    c_out.at[r, c, pl.ds(row0, RPT), :])]
        ftail = [(acc.at[pl.ds(RPT * NS, TAIL), :],
                  s_out.at[r, c, pl.ds(RPT * NS, TAIL), :]),
                 (cacc.at[pl.ds(RPT * NS, TAIL), :],
                  c_out.at[r, c, pl.ds(RPT * NS, TAIL), :])]
        for src, dst in fcopies:
            pltpu.async_copy(src, dst, sem_z)

        @pl.when(s == 0)
        def _flush_tail():
            for src, dst in ftail:
                pltpu.async_copy(src, dst, sem_z)

        for src, dst in fcopies:
            pltpu.make_async_copy(src, dst, sem_z).wait()

        @pl.when(s == 0)
        def _flush_tail_wait():
            for src, dst in ftail:
                pltpu.make_async_copy(src, dst, sem_z).wait()

        if r + 1 < nr:
            zero_accs()


def _make_sc(srcs):
    nr = len(srcs)
    return pl.kernel(
        functools.partial(_sc_body, srcs),
        out_type=(
            jax.ShapeDtypeStruct((nr, NC, N, D), jnp.float32),
            jax.ShapeDtypeStruct((nr, NC, N, CW), jnp.float32),
        ),
        mesh=plsc.VectorSubcoreMesh(
            core_axis_name="c", subcore_axis_name="s",
            num_cores=NC, num_subcores=NS),
        scratch_types=(
            [pltpu.VMEM_SHARED((N, D), jnp.float32),
             pltpu.VMEM_SHARED((N, CW), jnp.float32)]
            + [pltpu.VMEM((K, D), jnp.float32)] * 2
            + [pltpu.VMEM((2, K), jnp.int32)] * 4
            + [pltpu.VMEM((K, CW), jnp.float32)]
            + [pltpu.SemaphoreType.DMA] * 7
        ),
        compiler_params=pltpu.CompilerParams(use_tc_tiling_on_sc=False),
    )


_sc_v = _make_sc((0, 1, 2))
_sc_e = _make_sc((0, 2))
_sc_f = _make_sc((0, 1))


def _tc_combine(x, a, b, s_part, c_part, rs, wls):
    nrel = len(rs)
    blk = 1000
    ngrid = N // blk

    def body(*refs):
        x_ref = refs[0]
        a_ref = refs[1]
        b_ref = refs[2]
        c_ref = refs[3 + 3 * nrel]
        out_ref = refs[4 + 3 * nrel]
        cbuf = refs[5 + 3 * nrel]
        csem = refs[6 + 3 * nrel]
        i = pl.program_id(0)

        def fetch(gi, pb):
            for ri, r in enumerate(rs):
                for cc in (0, 1):
                    pltpu.make_async_copy(
                        c_ref.at[r, cc, pl.ds(gi * blk, blk), :],
                        cbuf.at[pb, ri, cc], csem).start()

        def drain(gi, pb):
            for ri, r in enumerate(rs):
                for cc in (0, 1):
                    pltpu.make_async_copy(
                        c_ref.at[r, cc, pl.ds(gi * blk, blk), :],
                        cbuf.at[pb, ri, cc], csem).wait()

        @pl.when(i == 0)
        def _():
            fetch(0, 0)

        drain(i, lax.rem(i, 2))

        @pl.when(i + 1 < ngrid)
        def _():
            fetch(i + 1, lax.rem(i + 1, 2))

        out = jnp.dot(x_ref[...], a_ref[...],
                      preferred_element_type=jnp.float32) + b_ref[...]
        pb = lax.rem(i, 2)
        for ri in range(nrel):
            s0 = refs[3 + 2 * ri][0, 0]
            s1 = refs[4 + 2 * ri][0, 0]
            wl = refs[3 + 2 * nrel + ri][...]
            cnt = cbuf[pb, ri, 0, :, 0:1] + cbuf[pb, ri, 1, :, 0:1]
            mean = (s0 + s1) * (1.0 / jnp.maximum(cnt, 1.0))
            out = out + jnp.dot(mean, wl, preferred_element_type=jnp.float32)
        out_ref[...] = jnp.maximum(out, 0.0)

    row_spec = pl.BlockSpec((blk, D), lambda i: (i, 0))
    w_spec = pl.BlockSpec((D, D), lambda i: (0, 0))
    b_spec = pl.BlockSpec((1, D), lambda i: (0, 0))
    in_specs = [row_spec, w_spec, b_spec]
    operands = [x, a, b]
    for r in rs:
        for cc in (0, 1):
            in_specs.append(pl.BlockSpec(
                (1, 1, blk, D), lambda i, r=r, cc=cc: (r, cc, i, 0)))
            operands.append(s_part)
    in_specs += [w_spec] * nrel
    operands += list(wls)
    in_specs.append(pl.BlockSpec(memory_space=pl.ANY))
    operands.append(c_part)
    return pl.pallas_call(
        body,
        grid=(ngrid,),
        in_specs=in_specs,
        out_specs=row_spec,
        out_shape=jax.ShapeDtypeStruct((N, D), jnp.float32),
        scratch_shapes=[
            pltpu.VMEM((2, nrel, 2, blk, CW), jnp.float32),
            pltpu.SemaphoreType.DMA,
        ],
    )(*operands)


def kernel(x_v, x_e, x_f,
           ei_vv, wl_vv, bl_vv, wr_vv,
           ei_ve, wl_ve, bl_ve, wr_ve,
           ei_vf, wl_vf, bl_vf, wr_vf,
           ei_ev, wl_ev, bl_ev, wr_ev,
           ei_ef, wl_ef, bl_ef, wr_ef,
           ei_fv, wl_fv, bl_fv, wr_fv,
           ei_fe, wl_fe, bl_fe, wr_fe,
           w_self_v, b_self_v,
           w_self_e, b_self_e,
           w_self_f, b_self_f):
    zf = jnp.zeros((RPT + 16, D), jnp.float32)
    zch = jnp.zeros((RPT + 16, CW), jnp.float32)
    s_v, c_v = _sc_v(x_v, x_e, x_f, ei_vv, ei_ev, ei_fv, zf, zch)
    s_e, c_e = _sc_e(x_v, x_e, x_f, ei_ve, ei_fe, zf, zch)
    s_f, c_f = _sc_f(x_v, x_e, x_f, ei_vf, ei_ef, zf, zch)

    a_v = w_self_v + (wr_vv + wr_ev + wr_fv) / 3.0
    b_v = (b_self_v + (bl_vv + bl_ev + bl_fv) / 3.0).reshape(1, D)
    a_e = w_self_e + (wr_ve + wr_fe) / 2.0
    b_e = (b_self_e + (bl_ve + bl_fe) / 2.0).reshape(1, D)
    a_f = w_self_f + (wr_vf + wr_ef) / 2.0
    b_f = (b_self_f + (bl_vf + bl_ef) / 2.0).reshape(1, D)

    xv = _tc_combine(x_v, a_v, b_v, s_v, c_v, [0, 1, 2],
                     [wl_vv / 3.0, wl_ev / 3.0, wl_fv / 3.0])
    xe = _tc_combine(x_e, a_e, b_e, s_e, c_e, [0, 1],
                     [wl_ve / 2.0, wl_fe / 2.0])
    xf = _tc_combine(x_f, a_f, b_f, s_f, c_f, [0, 1],
                     [wl_vf / 2.0, wl_ef / 2.0])
    return xv, xe, xf


# speedup vs baseline: 1.0221x; 1.0050x over previous
import functools

import jax
import jax.numpy as jnp
from jax import lax
from jax.experimental import pallas as pl
from jax.experimental.pallas import tpu as pltpu
from jax.experimental.pallas import tpu_sc as plsc

N = 10000
D = 128
E = 320000
NC = 2
NS = 16
K = 128
EPC = E // NC
EPT_M = 9984
MQ = 76
LEFT0 = NS * EPT_M
RPT = 624
TAIL = N - RPT * NS
CW = 16


def _sc_body(srcs, *args):
    nr = len(srcs)
    xv, xe, xf = args[0:3]
    es = args[3:3 + nr]
    zf, zch, s_out, c_out = args[3 + nr:7 + nr]
    (acc, cacc, rows0, rows1, i20, i21, i22, i23, ones,
     sem_g0, sem_g1, sem_i0, sem_i1, sem_s0, sem_s1, sem_z) = args[7 + nr:]
    c = lax.axis_index("c")
    s = lax.axis_index("s")
    one0 = jnp.where(lax.iota(jnp.int32, 16) == 0, 1.0, 0.0)
    rows = (rows0, rows1)
    i2 = (i20, i21, i22, i23)
    sem_g = (sem_g0, sem_g1)
    sem_i = (sem_i0, sem_i1)
    sem_s = (sem_s0, sem_s1)

    def init_ones(i, carry):
        ones[i, :] = one0
        return carry

    lax.fori_loop(0, K, init_ones, 0)

    tables = [(xv, xe, xf)[t] for t in srcs]
    edges = list(es)
    row0 = s * RPT
    ebase = c * EPC + s * EPT_M

    def zero_accs():
        zcopies = [(zf.at[pl.ds(0, RPT), :], acc.at[pl.ds(row0, RPT), :]),
                   (zch.at[pl.ds(0, RPT), :], cacc.at[pl.ds(row0, RPT), :])]
        tcopies = [(zf.at[pl.ds(0, TAIL), :],
                    acc.at[pl.ds(RPT * NS, TAIL), :]),
                   (zch.at[pl.ds(0, TAIL), :],
                    cacc.at[pl.ds(RPT * NS, TAIL), :])]
        for src, dst in zcopies:
            pltpu.async_copy(src, dst, sem_z)

        @pl.when(s == 0)
        def _():
            for src, dst in tcopies:
                pltpu.async_copy(src, dst, sem_z)

        for src, dst in zcopies:
            pltpu.make_async_copy(src, dst, sem_z).wait()

        @pl.when(s == 0)
        def _():
            for src, dst in tcopies:
                pltpu.make_async_copy(src, dst, sem_z).wait()

    def wait_scat(rb):
        pltpu.make_async_copy(rows[rb], acc.at[i20.at[1]], sem_s[rb]).wait()
        pltpu.make_async_copy(ones, cacc.at[i20.at[1]], sem_s[rb]).wait()

    def scat(rb, db):
        pltpu.async_copy(rows[rb], acc.at[i2[db].at[1]], sem_s[rb],
                         add=True)
        pltpu.async_copy(ones, cacc.at[i2[db].at[1]], sem_s[rb], add=True)

    def two_chunks(table, ei, base0, base1):
        pltpu.sync_copy(ei.at[:, pl.ds(base0, K)], i20)
        pltpu.async_copy(table.at[i20.at[0]], rows0, sem_g0)
        pltpu.sync_copy(ei.at[:, pl.ds(base1, K)], i21)
        pltpu.make_async_copy(table.at[i20.at[0]], rows0, sem_g0).wait()
        pltpu.async_copy(table.at[i21.at[0]], rows1, sem_g1)
        scat(0, 0)
        pltpu.make_async_copy(table.at[i21.at[0]], rows1, sem_g1).wait()
        scat(1, 1)
        wait_scat(0)
        wait_scat(1)

    def prologue(r):
        table = tables[r]
        ei = edges[r]
        pltpu.sync_copy(ei.at[:, pl.ds(ebase, K)], i20)
        pltpu.async_copy(table.at[i20.at[0]], rows[0], sem_g[0])
        pltpu.async_copy(ei.at[:, pl.ds(ebase + K, K)], i21, sem_i[1])

    zero_accs()
    prologue(0)

    for r in range(nr):
        table = tables[r]
        ei = edges[r]
        e0 = ebase

        def wait_idx(rb, ei=ei):
            pltpu.make_async_copy(
                ei.at[:, pl.ds(e0, K)], i20, sem_i[rb]).wait()

        plsc.subcore_barrier()


        def quad(p, carry):
            for b4 in range(4):
                g = p * 4 + b4
                rb = b4 % 2
                ob = 1 - rb
                pltpu.make_async_copy(
                    table.at[i2[b4].at[0]], rows[rb], sem_g[rb]).wait()

                @pl.when((g + 1 < MQ) & (g >= 1))
                def _():
                    wait_scat(ob)

                @pl.when(g + 1 < MQ)
                def _():
                    wait_idx(ob)
                    pltpu.async_copy(table.at[i2[(b4 + 1) % 4].at[0]],
                                     rows[ob], sem_g[ob])

                scat(rb, b4)

                @pl.when(g + 2 < MQ)
                def _():
                    b2 = e0 + (g + 2) * K
                    pltpu.async_copy(ei.at[:, pl.ds(b2, K)],
                                     i2[(b4 + 2) % 4], sem_i[rb])
            return carry

        lax.fori_loop(0, MQ // 4, quad, 0)
        wait_scat(0)
        wait_scat(1)

        two_chunks(table, ei, e0 + MQ * K, e0 + (MQ + 1) * K)

        @pl.when(s == 0)
        def _leftover():
            bl = c * EPC + LEFT0
            two_chunks(table, ei, bl, bl + K)

        if r + 1 < nr:
            prologue(r + 1)

        plsc.subcore_barrier()

        fcopies = [(acc.at[pl.ds(row0, RPT), :],
                    s_out.at[r, c, pl.ds(row0, RPT), :]),
                   (cacc.at[pl.ds(row0, RPT), :],
                    c_out.at[r, c, pl.ds(row0, RPT), :])]
        ftail = [(acc.at[pl.ds(RPT * NS, TAIL), :],
                  s_out.at[r, c, pl.ds(RPT * NS, TAIL), :]),
                 (cacc.at[pl.ds(RPT * NS, TAIL), :],
                  c_out.at[r, c, pl.ds(RPT * NS, TAIL), :])]
        for src, dst in fcopies:
            pltpu.async_copy(src, dst, sem_z)

        @pl.when(s == 0)
        def _flush_tail():
            for src, dst in ftail:
                pltpu.async_copy(src, dst, sem_z)

        for src, dst in fcopies:
            pltpu.make_async_copy(src, dst, sem_z).wait()

        @pl.when(s == 0)
        def _flush_tail_wait():
            for src, dst in ftail:
                pltpu.make_async_copy(src, dst, sem_z).wait()

        if r + 1 < nr:
            zero_accs()


def _make_sc(srcs):
    nr = len(srcs)
    return pl.kernel(
        functools.partial(_sc_body, srcs),
        out_type=(
            jax.ShapeDtypeStruct((nr, NC, N, D), jnp.float32),
            jax.ShapeDtypeStruct((nr, NC, N, CW), jnp.float32),
        ),
        mesh=plsc.VectorSubcoreMesh(
            core_axis_name="c", subcore_axis_name="s",
            num_cores=NC, num_subcores=NS),
        scratch_types=(
            [pltpu.VMEM_SHARED((N, D), jnp.float32),
             pltpu.VMEM_SHARED((N, CW), jnp.float32)]
            + [pltpu.VMEM((K, D), jnp.float32)] * 2
            + [pltpu.VMEM((2, K), jnp.int32)] * 4
            + [pltpu.VMEM((K, CW), jnp.float32)]
            + [pltpu.SemaphoreType.DMA] * 7
        ),
        compiler_params=pltpu.CompilerParams(use_tc_tiling_on_sc=False),
    )


_sc_v = _make_sc((0, 1, 2))
_sc_e = _make_sc((0, 2))
_sc_f = _make_sc((0, 1))


def _tc_combine(x, a, b, s_part, c_part, rs, wls):
    nrel = len(rs)
    blk = 1000
    ngrid = N // blk

    def body(*refs):
        x_ref = refs[0]
        a_ref = refs[1]
        b_ref = refs[2]
        c_ref = refs[3 + 3 * nrel]
        out_ref = refs[4 + 3 * nrel]
        cbuf = refs[5 + 3 * nrel]
        csem = refs[6 + 3 * nrel]
        i = pl.program_id(0)

        def fetch(gi, pb):
            for ri, r in enumerate(rs):
                for cc in (0, 1):
                    pltpu.make_async_copy(
                        c_ref.at[r, cc, pl.ds(gi * blk, blk), :],
                        cbuf.at[pb, ri, cc], csem).start()

        def drain(gi, pb):
            for ri, r in enumerate(rs):
                for cc in (0, 1):
                    pltpu.make_async_copy(
                        c_ref.at[r, cc, pl.ds(gi * blk, blk), :],
                        cbuf.at[pb, ri, cc], csem).wait()

        @pl.when(i == 0)
        def _():
            fetch(0, 0)

        drain(i, lax.rem(i, 2))

        @pl.when(i + 1 < ngrid)
        def _():
            fetch(i + 1, lax.rem(i + 1, 2))

        out = jnp.dot(x_ref[...], a_ref[...],
                      preferred_element_type=jnp.float32) + b_ref[...]
        pb = lax.rem(i, 2)
        for ri in range(nrel):
            s0 = refs[3 + 2 * ri][0, 0]
            s1 = refs[4 + 2 * ri][0, 0]
            wl = refs[3 + 2 * nrel + ri][...]
            cnt = cbuf[pb, ri, 0, :, 0:1] + cbuf[pb, ri, 1, :, 0:1]
            mean = (s0 + s1) * (1.0 / jnp.maximum(cnt, 1.0))
            out = out + jnp.dot(mean, wl, preferred_element_type=jnp.float32)
        out_ref[...] = jnp.maximum(out, 0.0)

    row_spec = pl.BlockSpec((blk, D), lambda i: (i, 0))
    w_spec = pl.BlockSpec((D, D), lambda i: (0, 0))
    b_spec = pl.BlockSpec((1, D), lambda i: (0, 0))
    in_specs = [row_spec, w_spec, b_spec]
    operands = [x, a, b]
    for r in rs:
        for cc in (0, 1):
            in_specs.append(pl.BlockSpec(
                (1, 1, blk, D), lambda i, r=r, cc=cc: (r, cc, i, 0)))
            operands.append(s_part)
    in_specs += [w_spec] * nrel
    operands += list(wls)
    in_specs.append(pl.BlockSpec(memory_space=pl.ANY))
    operands.append(c_part)
    return pl.pallas_call(
        body,
        grid=(ngrid,),
        in_specs=in_specs,
        out_specs=row_spec,
        out_shape=jax.ShapeDtypeStruct((N, D), jnp.float32),
        scratch_shapes=[
            pltpu.VMEM((2, nrel, 2, blk, CW), jnp.float32),
            pltpu.SemaphoreType.DMA,
        ],
    )(*operands)


def kernel(x_v, x_e, x_f,
           ei_vv, wl_vv, bl_vv, wr_vv,
           ei_ve, wl_ve, bl_ve, wr_ve,
           ei_vf, wl_vf, bl_vf, wr_vf,
           ei_ev, wl_ev, bl_ev, wr_ev,
           ei_ef, wl_ef, bl_ef, wr_ef,
           ei_fv, wl_fv, bl_fv, wr_fv,
           ei_fe, wl_fe, bl_fe, wr_fe,
           w_self_v, b_self_v,
           w_self_e, b_self_e,
           w_self_f, b_self_f):
    zf = jnp.zeros((RPT + 16, D), jnp.float32)
    zch = jnp.zeros((RPT + 16, CW), jnp.float32)
    s_v, c_v = _sc_v(x_v, x_e, x_f, ei_vv, ei_ev, ei_fv, zf, zch)
    s_e, c_e = _sc_e(x_v, x_e, x_f, ei_ve, ei_fe, zf, zch)
    s_f, c_f = _sc_f(x_v, x_e, x_f, ei_vf, ei_ef, zf, zch)

    a_v = w_self_v + (wr_vv + wr_ev + wr_fv) / 3.0
    b_v = (b_self_v + (bl_vv + bl_ev + bl_fv) / 3.0).reshape(1, D)
    a_e = w_self_e + (wr_ve + wr_fe) / 2.0
    b_e = (b_self_e + (bl_ve + bl_fe) / 2.0).reshape(1, D)
    a_f = w_self_f + (wr_vf + wr_ef) / 2.0
    b_f = (b_self_f + (bl_vf + bl_ef) / 2.0).reshape(1, D)

    xv = _tc_combine(x_v, a_v, b_v, s_v, c_v, [0, 1, 2],
                     [wl_vv / 3.0, wl_ev / 3.0, wl_fv / 3.0])
    xe = _tc_combine(x_e, a_e, b_e, s_e, c_e, [0, 1],
                     [wl_ve / 2.0, wl_fe / 2.0])
    xf = _tc_combine(x_f, a_f, b_f, s_f, c_f, [0, 1],
                     [wl_vf / 2.0, wl_ef / 2.0])
    return xv, xe, xf
